# baseline (speedup 1.0000x reference)
import jax
import jax.numpy as jnp
from jax.experimental import pallas as pl
from jax.experimental.pallas import tpu as pltpu

_NUM_CLASSES = 19
_IGNORE_INDEX = 255
_LANES = 128
_CBLK = 128


def _pool_kernel(gt_ref, f_ref, o_ref, onehot_ref, recip_ref):
    j = pl.program_id(1)
    hw = onehot_ref.shape[0]

    @pl.when(j == 0)
    def _():
        gt = gt_ref[0]
        valid = gt != _IGNORE_INDEX
        cls = jnp.clip(gt, 0, _NUM_CLASSES - 1)
        lane = jax.lax.broadcasted_iota(jnp.int32, (hw, _LANES), 1)
        onehot = ((cls == lane) & valid).astype(jnp.float32)
        onehot_ref[...] = onehot
        cnt = jnp.sum(onehot, axis=0, keepdims=True)
        recip_ref[...] = 1.0 / jnp.where(cnt > 0.0, cnt, 1.0)

    acc = jnp.dot(f_ref[0], onehot_ref[...],
                  preferred_element_type=jnp.float32)
    o_ref[0] = acc * recip_ref[...]


def kernel(feats, gt_seg_map):
    B, C, H, W = feats.shape
    HW = H * W
    f = feats.reshape(B, C, HW)
    gt = gt_seg_map.astype(jnp.int32).reshape(B, HW, 1)

    out = pl.pallas_call(
        _pool_kernel,
        grid=(B, C // _CBLK),
        in_specs=[
            pl.BlockSpec((1, HW, 1), lambda b, j: (b, 0, 0)),
            pl.BlockSpec((1, _CBLK, HW), lambda b, j: (b, j, 0)),
        ],
        out_specs=pl.BlockSpec((1, _CBLK, _LANES), lambda b, j: (b, j, 0)),
        out_shape=jax.ShapeDtypeStruct((B, C, _LANES), jnp.float32),
        scratch_shapes=[
            pltpu.VMEM((HW, _LANES), jnp.float32),
            pltpu.VMEM((1, _LANES), jnp.float32),
        ],
        compiler_params=pltpu.CompilerParams(
            dimension_semantics=("parallel", "arbitrary"),
            vmem_limit_bytes=56 * 1024 * 1024,
        ),
        name="class_mean_pool",
    )(gt, f)

    return out[:, :, :_NUM_CLASSES, None]


# speedup vs baseline: 2.0770x; 2.0770x over previous
import jax
import jax.numpy as jnp
from jax.experimental import pallas as pl
from jax.experimental.pallas import tpu as pltpu

_NUM_CLASSES = 19
_IGNORE_INDEX = 255
_LANES = 128
_CBLK = 128


def _pool_kernel(gt_ref, f_ref, o_ref, onehot_ref, recip_ref):
    j = pl.program_id(1)
    hw = onehot_ref.shape[0]

    @pl.when(j == 0)
    def _():
        gt = gt_ref[0]
        valid = gt != _IGNORE_INDEX
        cls = jnp.clip(gt, 0, _NUM_CLASSES - 1)
        lane = jax.lax.broadcasted_iota(jnp.int32, (hw, _LANES), 1)
        onehot = ((cls == lane) & valid).astype(jnp.float32)
        onehot_ref[...] = onehot
        cnt = jnp.sum(onehot, axis=0, keepdims=True)
        recip_ref[...] = 1.0 / jnp.where(cnt > 0.0, cnt, 1.0)

    f = f_ref[0].reshape(_CBLK, hw)
    acc = jnp.dot(f, onehot_ref[...],
                  preferred_element_type=jnp.float32)
    o_ref[0] = acc * recip_ref[...]


def kernel(feats, gt_seg_map):
    B, C, H, W = feats.shape
    HW = H * W
    gt = gt_seg_map.astype(jnp.int32).reshape(B, HW, 1)

    out = pl.pallas_call(
        _pool_kernel,
        grid=(B, C // _CBLK),
        in_specs=[
            pl.BlockSpec((1, HW, 1), lambda b, j: (b, 0, 0)),
            pl.BlockSpec((1, _CBLK, H, W), lambda b, j: (b, j, 0, 0)),
        ],
        out_specs=pl.BlockSpec((1, _CBLK, _LANES), lambda b, j: (b, j, 0)),
        out_shape=jax.ShapeDtypeStruct((B, C, _LANES), jnp.float32),
        scratch_shapes=[
            pltpu.VMEM((HW, _LANES), jnp.float32),
            pltpu.VMEM((1, _LANES), jnp.float32),
        ],
        compiler_params=pltpu.CompilerParams(
            dimension_semantics=("parallel", "arbitrary"),
            vmem_limit_bytes=56 * 1024 * 1024,
        ),
        name="class_mean_pool",
    )(gt, feats)

    return out[:, :, :_NUM_CLASSES, None]


# speedup vs baseline: 3.4049x; 1.6393x over previous
import jax
import jax.numpy as jnp
from jax.experimental import pallas as pl
from jax.experimental.pallas import tpu as pltpu

_NUM_CLASSES = 19
_IGNORE_INDEX = 255
_LANES = 128
_CBLK = 256


def _pool_kernel(gt_ref, f_ref, o_ref, wt_ref):
    j = pl.program_id(1)
    hw = wt_ref.shape[1]

    @pl.when(j == 0)
    def _():
        gt = gt_ref[0].reshape(1, hw)
        valid = gt != _IGNORE_INDEX
        cls = jnp.clip(gt, 0, _NUM_CLASSES - 1)
        row = jax.lax.broadcasted_iota(jnp.int32, (_LANES, hw), 0)
        onehot = ((cls == row) & valid).astype(jnp.float32)
        cnt = jnp.sum(onehot, axis=1, keepdims=True)
        wt_ref[...] = onehot / jnp.where(cnt > 0.0, cnt, 1.0)

    f = f_ref[0].reshape(f_ref.shape[1], hw)
    o_ref[0] = jax.lax.dot_general(
        f, wt_ref[...],
        dimension_numbers=(((1,), (1,)), ((), ())),
        preferred_element_type=jnp.float32,
    )


def kernel(feats, gt_seg_map):
    B, C, H, W = feats.shape
    HW = H * W
    gt = gt_seg_map.astype(jnp.int32)

    out = pl.pallas_call(
        _pool_kernel,
        grid=(B, C // _CBLK),
        in_specs=[
            pl.BlockSpec((1, H, W), lambda b, j: (b, 0, 0)),
            pl.BlockSpec((1, _CBLK, H, W), lambda b, j: (b, j, 0, 0)),
        ],
        out_specs=pl.BlockSpec((1, _CBLK, _LANES), lambda b, j: (b, j, 0)),
        out_shape=jax.ShapeDtypeStruct((B, C, _LANES), jnp.float32),
        scratch_shapes=[
            pltpu.VMEM((_LANES, HW), jnp.float32),
        ],
        compiler_params=pltpu.CompilerParams(
            dimension_semantics=("parallel", "arbitrary"),
            vmem_limit_bytes=56 * 1024 * 1024,
        ),
        name="class_mean_pool",
    )(gt, feats)

    return out[:, :, :_NUM_CLASSES, None]


# speedup vs baseline: 3.5920x; 1.0549x over previous
import jax
import jax.numpy as jnp
from jax.experimental import pallas as pl
from jax.experimental.pallas import tpu as pltpu

_NUM_CLASSES = 19
_IGNORE_INDEX = 255
_LANES = 128
_CBLK = 128


def _pool_kernel(gt_ref, f_ref, o_ref, wt_ref):
    j = pl.program_id(1)
    hw = wt_ref.shape[1]

    @pl.when(j == 0)
    def _():
        gt = gt_ref[0].reshape(1, hw)
        valid = gt != _IGNORE_INDEX
        cls = jnp.clip(gt, 0, _NUM_CLASSES - 1)
        row = jax.lax.broadcasted_iota(jnp.int32, (_LANES, hw), 0)
        onehot = ((cls == row) & valid).astype(jnp.float32)
        cnt = jnp.sum(onehot, axis=1, keepdims=True)
        wt_ref[...] = onehot / jnp.where(cnt > 0.0, cnt, 1.0)

    f = f_ref[0].reshape(f_ref.shape[1], hw)
    o_ref[0] = jax.lax.dot_general(
        f, wt_ref[...],
        dimension_numbers=(((1,), (1,)), ((), ())),
        preferred_element_type=jnp.float32,
    )


def kernel(feats, gt_seg_map):
    B, C, H, W = feats.shape
    HW = H * W
    gt = gt_seg_map.astype(jnp.int32)

    out = pl.pallas_call(
        _pool_kernel,
        grid=(B, C // _CBLK),
        in_specs=[
            pl.BlockSpec((1, H, W), lambda b, j: (b, 0, 0)),
            pl.BlockSpec((1, _CBLK, H, W), lambda b, j: (b, j, 0, 0)),
        ],
        out_specs=pl.BlockSpec((1, _CBLK, _LANES), lambda b, j: (b, j, 0)),
        out_shape=jax.ShapeDtypeStruct((B, C, _LANES), jnp.float32),
        scratch_shapes=[
            pltpu.VMEM((_LANES, HW), jnp.float32),
        ],
        compiler_params=pltpu.CompilerParams(
            dimension_semantics=("parallel", "arbitrary"),
            vmem_limit_bytes=56 * 1024 * 1024,
        ),
        name="class_mean_pool",
    )(gt, feats)

    return out[:, :, :_NUM_CLASSES, None]
